# baseline (speedup 1.0000x reference)
import functools

import jax
import jax.numpy as jnp
from jax import lax
from jax.experimental import pallas as pl
from jax.experimental.pallas import tpu as pltpu
from jax.experimental.pallas import tpu_sc as plsc

CHUNK = 100
K = 8


@functools.lru_cache(maxsize=None)
def _make_gather(b: int, s: int, vocab: int, d: int):
    info = plsc.get_sparse_core_info()
    nc, ns = info.num_cores, info.num_subcores
    nw = nc * ns
    assert s == 2 * CHUNK and b % nw == 0
    b_per_w = b // nw
    steps = b_per_w * 2
    assert steps % K == 0
    groups = steps // K
    gb = K // 2

    @functools.partial(
        pl.kernel,
        mesh=plsc.VectorSubcoreMesh(core_axis_name="c", subcore_axis_name="s"),
        out_type=jax.ShapeDtypeStruct((b, s, d), jnp.float32),
        scratch_types=[
            pltpu.VMEM((steps, CHUNK), jnp.int32),
            pltpu.VMEM((2, gb, s, d), jnp.float32),
            pltpu.SemaphoreType.DMA,
            pltpu.SemaphoreType.DMA,
        ],
        compiler_params=pltpu.CompilerParams(use_tc_tiling_on_sc=False),
    )
    def gather_kernel(idx_hbm, table_hbm, out_hbm, idx_v, rows_v, gsem, osem):
        wid = lax.axis_index("s") * nc + lax.axis_index("c")
        bbase = wid * b_per_w
        pltpu.sync_copy(idx_hbm.at[wid], idx_v)

        def fire(g, p):
            return [
                pltpu.async_copy(
                    table_hbm.at[idx_v.at[g * K + j]],
                    rows_v.at[p].at[j // 2, pl.ds((j % 2) * CHUNK, CHUNK)],
                    gsem,
                )
                for j in range(K)
            ]

        def start_out(g, p):
            return pltpu.async_copy(
                rows_v.at[p], out_hbm.at[pl.ds(bbase + g * gb, gb)], osem
            )

        for d_ in fire(0, 0):
            d_.wait()

        def body(i, carry):
            p = i % 2
            od = start_out(i, p)
            gds = fire(i + 1, 1 - p)
            for d_ in gds:
                d_.wait()
            od.wait()
            return carry

        lax.fori_loop(0, groups - 1, body, 0)
        start_out(groups - 1, (groups - 1) % 2).wait()

    return gather_kernel


def kernel(gene_ids, table):
    b, s = gene_ids.shape
    vocab, d = table.shape
    info = plsc.get_sparse_core_info()
    nw = info.num_cores * info.num_subcores
    idx = gene_ids.reshape(nw, (b * s) // (nw * CHUNK), CHUNK).astype(jnp.int32)
    out = _make_gather(b, s, vocab, d)(idx, table)
    return gene_ids, out
